# Initial kernel scaffold; baseline (speedup 1.0000x reference)
import functools

import jax
import jax.numpy as jnp
from jax import lax
from jax.experimental import pallas as pl
from jax.experimental.pallas import tpu as pltpu
from jax.experimental.pallas import tpu_sc as plsc

PROJ = 128
GRID = 32
NPTS = 16 * 16384
NW = 32
PPW = NPTS // NW
K = 32
NCH = PPW // K
L = 16

OFFS = tuple(dz * 1024 + dy * 32 + dx
             for dz in (0, 1) for dy in (0, 1) for dx in (0, 1))


def _axis_iw(c):
    ic = jnp.clip((c + 1.0) * jnp.float32(0.5 * (GRID - 1)), 0.0,
                  jnp.float32(GRID - 1))
    i0 = jnp.minimum(ic.astype(jnp.int32), GRID - 2)
    f1 = ic - i0.astype(jnp.float32)
    return i0, f1


def _sc_body(cx_hbm, cy_hbm, cz_hbm, table_hbm, out_hbm,
             cxv, cyv, czv, idx_v, w_v, rows_v, out_v, sem):
    cid = lax.axis_index("c")
    sid = lax.axis_index("s")
    wid = sid * 2 + cid
    base_pt = wid * PPW

    pltpu.sync_copy(cx_hbm.at[pl.ds(base_pt, PPW)], cxv)
    pltpu.sync_copy(cy_hbm.at[pl.ds(base_pt, PPW)], cyv)
    pltpu.sync_copy(cz_hbm.at[pl.ds(base_pt, PPW)], czv)

    def chunk_body(g, carry):
        cb = g * K
        for s in range(K // L):
            o = cb + s * L
            x0, fx1 = _axis_iw(cxv[pl.ds(o, L)])
            y0, fy1 = _axis_iw(cyv[pl.ds(o, L)])
            z0, fz1 = _axis_iw(czv[pl.ds(o, L)])
            base = z0 * (GRID * GRID) + y0 * GRID + x0
            fx0 = 1.0 - fx1
            fy0 = 1.0 - fy1
            fz0 = 1.0 - fz1
            for c in range(8):
                dz, dy, dx = (c >> 2) & 1, (c >> 1) & 1, c & 1
                idx_v[c, pl.ds(s * L, L)] = base + OFFS[c]
                w_v[c, pl.ds(s * L, L)] = ((fz1 if dz else fz0)
                                           * (fy1 if dy else fy0)
                                           * (fx1 if dx else fx0))
        copies = [pltpu.async_copy(table_hbm.at[idx_v.at[c]], rows_v.at[c], sem)
                  for c in range(8)]
        for cp in copies:
            cp.wait()

        def pt_body(p, _):
            w = [w_v[c, p] for c in range(8)]
            for v in range(PROJ // L):
                a = w[0] * rows_v[0, p, pl.ds(v * L, L)]
                for c in range(1, 8):
                    a = a + w[c] * rows_v[c, p, pl.ds(v * L, L)]
                out_v[p, pl.ds(v * L, L)] = a
            return 0

        lax.fori_loop(0, K, pt_body, 0, unroll=False)
        pltpu.sync_copy(out_v, out_hbm.at[pl.ds(base_pt + cb, K)])
        return carry

    lax.fori_loop(0, NCH, chunk_body, 0, unroll=False)


@functools.partial(
    pl.kernel,
    out_type=jax.ShapeDtypeStruct((NPTS, PROJ), jnp.float32),
    mesh=plsc.VectorSubcoreMesh(core_axis_name="c", subcore_axis_name="s"),
    scratch_types=[
        pltpu.VMEM((PPW,), jnp.float32),
        pltpu.VMEM((PPW,), jnp.float32),
        pltpu.VMEM((PPW,), jnp.float32),
        pltpu.VMEM((8, K), jnp.int32),
        pltpu.VMEM((8, K), jnp.float32),
        pltpu.VMEM((8, K, PROJ), jnp.float32),
        pltpu.VMEM((K, PROJ), jnp.float32),
        pltpu.SemaphoreType.DMA,
    ],
)
def _trilerp_sc(cx_hbm, cy_hbm, cz_hbm, table_hbm, out_hbm, *scratch):
    _sc_body(cx_hbm, cy_hbm, cz_hbm, table_hbm, out_hbm, *scratch)


def kernel(coordinates, pos_enc):
    B, N, _ = coordinates.shape
    ct = coordinates.reshape(B * N, 3)
    cx, cy, cz = ct[:, 2], ct[:, 1], ct[:, 0]
    table = jnp.transpose(pos_enc[0], (1, 2, 3, 0)).reshape(GRID ** 3, PROJ)
    out = _trilerp_sc(cx, cy, cz, table)
    return out.reshape(B, N, PROJ)


# speedup vs baseline: 19.9617x; 19.9617x over previous
import functools

import jax
import jax.numpy as jnp
from jax import lax
from jax.experimental import pallas as pl
from jax.experimental.pallas import tpu as pltpu
from jax.experimental.pallas import tpu_sc as plsc

PROJ = 128
GRID = 32
NPTS = 16 * 16384
NW = 32
PPW = NPTS // NW
K = 32
NCH = PPW // K
L = 16

OFFS = tuple(dz * 1024 + dy * 32 + dx
             for dz in (0, 1) for dy in (0, 1) for dx in (0, 1))


def _axis_iw(c):
    ic = jnp.clip((c + 1.0) * jnp.float32(0.5 * (GRID - 1)), 0.0,
                  jnp.float32(GRID - 1))
    i0 = jnp.minimum(ic.astype(jnp.int32), GRID - 2)
    f1 = ic - i0.astype(jnp.float32)
    return i0, f1


def _sc_body(cx_hbm, cy_hbm, cz_hbm, table_hbm, out_hbm,
             cxv, cyv, czv, idx_v, w_v, rows_v, out_v, sem):
    cid = lax.axis_index("c")
    sid = lax.axis_index("s")
    wid = sid * 2 + cid
    base_pt = wid * PPW

    pltpu.sync_copy(cx_hbm.at[pl.ds(base_pt, PPW)], cxv)
    pltpu.sync_copy(cy_hbm.at[pl.ds(base_pt, PPW)], cyv)
    pltpu.sync_copy(cz_hbm.at[pl.ds(base_pt, PPW)], czv)

    def chunk_body(g, carry):
        cb = g * K
        for s in range(K // L):
            o = cb + s * L
            x0, fx1 = _axis_iw(cxv[pl.ds(o, L)])
            y0, fy1 = _axis_iw(cyv[pl.ds(o, L)])
            z0, fz1 = _axis_iw(czv[pl.ds(o, L)])
            base = z0 * (GRID * GRID) + y0 * GRID + x0
            fx0 = 1.0 - fx1
            fy0 = 1.0 - fy1
            fz0 = 1.0 - fz1
            for c in range(8):
                dz, dy, dx = (c >> 2) & 1, (c >> 1) & 1, c & 1
                idx_v[c, pl.ds(s * L, L)] = base + OFFS[c]
                w_v[c, pl.ds(s * L, L)] = ((fz1 if dz else fz0)
                                           * (fy1 if dy else fy0)
                                           * (fx1 if dx else fx0))
        copies = [pltpu.async_copy(table_hbm.at[idx_v.at[c]], rows_v.at[c], sem)
                  for c in range(8)]
        for cp in copies:
            cp.wait()

        def grp_body(s2, _):
            pb = s2 * L
            wvs = [w_v[c, pl.ds(pb, L)] for c in range(8)]
            for i in range(L):
                p = pb + i
                for v in range(PROJ // L):
                    a = wvs[0][i] * rows_v[0, p, pl.ds(v * L, L)]
                    for c in range(1, 8):
                        a = a + wvs[c][i] * rows_v[c, p, pl.ds(v * L, L)]
                    out_v[p, pl.ds(v * L, L)] = a
            return 0

        lax.fori_loop(0, K // L, grp_body, 0, unroll=False)
        pltpu.sync_copy(out_v, out_hbm.at[pl.ds(base_pt + cb, K)])
        return carry

    lax.fori_loop(0, NCH, chunk_body, 0, unroll=False)


@functools.partial(
    pl.kernel,
    out_type=jax.ShapeDtypeStruct((NPTS, PROJ), jnp.float32),
    mesh=plsc.VectorSubcoreMesh(core_axis_name="c", subcore_axis_name="s"),
    scratch_types=[
        pltpu.VMEM((PPW,), jnp.float32),
        pltpu.VMEM((PPW,), jnp.float32),
        pltpu.VMEM((PPW,), jnp.float32),
        pltpu.VMEM((8, K), jnp.int32),
        pltpu.VMEM((8, K), jnp.float32),
        pltpu.VMEM((8, K, PROJ), jnp.float32),
        pltpu.VMEM((K, PROJ), jnp.float32),
        pltpu.SemaphoreType.DMA,
    ],
)
def _trilerp_sc(cx_hbm, cy_hbm, cz_hbm, table_hbm, out_hbm, *scratch):
    _sc_body(cx_hbm, cy_hbm, cz_hbm, table_hbm, out_hbm, *scratch)


def kernel(coordinates, pos_enc):
    B, N, _ = coordinates.shape
    ct = coordinates.reshape(B * N, 3)
    cx, cy, cz = ct[:, 2], ct[:, 1], ct[:, 0]
    table = jnp.transpose(pos_enc[0], (1, 2, 3, 0)).reshape(GRID ** 3, PROJ)
    out = _trilerp_sc(cx, cy, cz, table)
    return out.reshape(B, N, PROJ)


# speedup vs baseline: 24.1822x; 1.2114x over previous
import functools

import jax
import jax.numpy as jnp
from jax import lax
from jax.experimental import pallas as pl
from jax.experimental.pallas import tpu as pltpu
from jax.experimental.pallas import tpu_sc as plsc

PROJ = 128
GRID = 32
NPTS = 16 * 16384
NW = 32
PPW = NPTS // NW
K = 32
NCH = PPW // K
L = 16

OFFS = tuple(dz * 1024 + dy * 32 + dx
             for dz in (0, 1) for dy in (0, 1) for dx in (0, 1))


def _axis_iw(c):
    ic = jnp.clip((c + 1.0) * jnp.float32(0.5 * (GRID - 1)), 0.0,
                  jnp.float32(GRID - 1))
    i0 = jnp.minimum(ic.astype(jnp.int32), GRID - 2)
    f1 = ic - i0.astype(jnp.float32)
    return i0, f1


def _sc_body(cx_hbm, cy_hbm, cz_hbm, table_hbm, out_hbm,
             cxv, cyv, czv, idx_v, w_v, rows_v, out_v, sem0, sem1):
    cid = lax.axis_index("c")
    sid = lax.axis_index("s")
    wid = sid * 2 + cid
    base_pt = wid * PPW
    sems = (sem0, sem1)

    pltpu.sync_copy(cx_hbm.at[pl.ds(base_pt, PPW)], cxv)
    pltpu.sync_copy(cy_hbm.at[pl.ds(base_pt, PPW)], cyv)
    pltpu.sync_copy(cz_hbm.at[pl.ds(base_pt, PPW)], czv)

    def fire(g, b):
        cb = g * K
        for s in range(K // L):
            o = cb + s * L
            x0, fx1 = _axis_iw(cxv[pl.ds(o, L)])
            y0, fy1 = _axis_iw(cyv[pl.ds(o, L)])
            z0, fz1 = _axis_iw(czv[pl.ds(o, L)])
            base = z0 * (GRID * GRID) + y0 * GRID + x0
            fx0 = 1.0 - fx1
            fy0 = 1.0 - fy1
            fz0 = 1.0 - fz1
            for c in range(8):
                dz, dy, dx = (c >> 2) & 1, (c >> 1) & 1, c & 1
                idx_v[b, c, pl.ds(s * L, L)] = base + OFFS[c]
                w_v[b, c, pl.ds(s * L, L)] = ((fz1 if dz else fz0)
                                              * (fy1 if dy else fy0)
                                              * (fx1 if dx else fx0))
        for c in range(8):
            pltpu.async_copy(table_hbm.at[idx_v.at[b, c]], rows_v.at[b, c],
                             sems[b])

    def drain(b):
        for c in range(8):
            pltpu.make_async_copy(table_hbm.at[idx_v.at[b, c]],
                                  rows_v.at[b, c], sems[b]).wait()

    def combine_store(g, b):
        def grp_body(s2, _):
            pb = s2 * L
            wvs = [w_v[b, c, pl.ds(pb, L)] for c in range(8)]
            for i in range(L):
                p = pb + i
                for v in range(PROJ // L):
                    a = wvs[0][i] * rows_v[b, 0, p, pl.ds(v * L, L)]
                    for c in range(1, 8):
                        a = a + wvs[c][i] * rows_v[b, c, p, pl.ds(v * L, L)]
                    out_v[p, pl.ds(v * L, L)] = a
            return 0

        lax.fori_loop(0, K // L, grp_body, 0, unroll=False)
        pltpu.sync_copy(out_v, out_hbm.at[pl.ds(base_pt + g * K, K)])

    fire(0, 0)

    def body(h, carry):
        g0 = 2 * h
        fire(g0 + 1, 1)
        drain(0)
        combine_store(g0, 0)

        @pl.when(g0 + 2 < NCH)
        def _():
            fire(g0 + 2, 0)

        drain(1)
        combine_store(g0 + 1, 1)
        return carry

    lax.fori_loop(0, NCH // 2, body, 0, unroll=False)


@functools.partial(
    pl.kernel,
    out_type=jax.ShapeDtypeStruct((NPTS, PROJ), jnp.float32),
    mesh=plsc.VectorSubcoreMesh(core_axis_name="c", subcore_axis_name="s"),
    scratch_types=[
        pltpu.VMEM((PPW,), jnp.float32),
        pltpu.VMEM((PPW,), jnp.float32),
        pltpu.VMEM((PPW,), jnp.float32),
        pltpu.VMEM((2, 8, K), jnp.int32),
        pltpu.VMEM((2, 8, K), jnp.float32),
        pltpu.VMEM((2, 8, K, PROJ), jnp.float32),
        pltpu.VMEM((K, PROJ), jnp.float32),
        pltpu.SemaphoreType.DMA,
        pltpu.SemaphoreType.DMA,
    ],
)
def _trilerp_sc(cx_hbm, cy_hbm, cz_hbm, table_hbm, out_hbm, *scratch):
    _sc_body(cx_hbm, cy_hbm, cz_hbm, table_hbm, out_hbm, *scratch)


def kernel(coordinates, pos_enc):
    B, N, _ = coordinates.shape
    ct = coordinates.reshape(B * N, 3)
    cx, cy, cz = ct[:, 2], ct[:, 1], ct[:, 0]
    table = jnp.transpose(pos_enc[0], (1, 2, 3, 0)).reshape(GRID ** 3, PROJ)
    out = _trilerp_sc(cx, cy, cz, table)
    return out.reshape(B, N, PROJ)


# speedup vs baseline: 24.5488x; 1.0152x over previous
import functools

import jax
import jax.numpy as jnp
from jax import lax
from jax.experimental import pallas as pl
from jax.experimental.pallas import tpu as pltpu
from jax.experimental.pallas import tpu_sc as plsc

PROJ = 128
GRID = 32
NPTS = 16 * 16384
NW = 32
PPW = NPTS // NW
K = 32
NCH = PPW // K
L = 16

OFFS = tuple(dz * 1024 + dy * 32 + dx
             for dz in (0, 1) for dy in (0, 1) for dx in (0, 1))


def _axis_iw(c):
    ic = jnp.clip((c + 1.0) * jnp.float32(0.5 * (GRID - 1)), 0.0,
                  jnp.float32(GRID - 1))
    i0 = jnp.minimum(ic.astype(jnp.int32), GRID - 2)
    f1 = ic - i0.astype(jnp.float32)
    return i0, f1


def _sc_body(cx_hbm, cy_hbm, cz_hbm, table_hbm, out_hbm,
             cxv, cyv, czv, idx_v, w_v, rows_v, out_v,
             sem0, sem1, semo0, semo1):
    cid = lax.axis_index("c")
    sid = lax.axis_index("s")
    wid = sid * 2 + cid
    base_pt = wid * PPW
    sems = (sem0, sem1)
    sems_out = (semo0, semo1)

    pltpu.sync_copy(cx_hbm.at[pl.ds(base_pt, PPW)], cxv)
    pltpu.sync_copy(cy_hbm.at[pl.ds(base_pt, PPW)], cyv)
    pltpu.sync_copy(cz_hbm.at[pl.ds(base_pt, PPW)], czv)

    def fire(g, b):
        cb = g * K
        for s in range(K // L):
            o = cb + s * L
            x0, fx1 = _axis_iw(cxv[pl.ds(o, L)])
            y0, fy1 = _axis_iw(cyv[pl.ds(o, L)])
            z0, fz1 = _axis_iw(czv[pl.ds(o, L)])
            base = z0 * (GRID * GRID) + y0 * GRID + x0
            fx0 = 1.0 - fx1
            fy0 = 1.0 - fy1
            fz0 = 1.0 - fz1
            for c in range(8):
                dz, dy, dx = (c >> 2) & 1, (c >> 1) & 1, c & 1
                idx_v[b, c, pl.ds(s * L, L)] = base + OFFS[c]
                w_v[b, c, pl.ds(s * L, L)] = ((fz1 if dz else fz0)
                                              * (fy1 if dy else fy0)
                                              * (fx1 if dx else fx0))
        for c in range(8):
            pltpu.async_copy(table_hbm.at[idx_v.at[b, c]], rows_v.at[b, c],
                             sems[b])

    def drain(b):
        for c in range(8):
            pltpu.make_async_copy(table_hbm.at[idx_v.at[b, c]],
                                  rows_v.at[b, c], sems[b]).wait()

    def combine_store(g, b, first):
        def grp_body(s2, _):
            pb = s2 * L
            wvs = [w_v[b, c, pl.ds(pb, L)] for c in range(8)]
            for i in range(L):
                p = pb + i
                for v in range(PROJ // L):
                    a = wvs[0][i] * rows_v[b, 0, p, pl.ds(v * L, L)]
                    for c in range(1, 8):
                        a = a + wvs[c][i] * rows_v[b, c, p, pl.ds(v * L, L)]
                    out_v[b, p, pl.ds(v * L, L)] = a
            return 0

        @pl.when(jnp.logical_not(first))
        def _():
            pltpu.make_async_copy(
                out_v.at[b], out_hbm.at[pl.ds(base_pt, K)],
                sems_out[b]).wait()

        lax.fori_loop(0, K // L, grp_body, 0, unroll=False)
        pltpu.async_copy(out_v.at[b], out_hbm.at[pl.ds(base_pt + g * K, K)],
                         sems_out[b])

    fire(0, 0)

    def body(h, carry):
        g0 = 2 * h
        first = h == 0
        fire(g0 + 1, 1)
        drain(0)
        combine_store(g0, 0, first)

        @pl.when(g0 + 2 < NCH)
        def _():
            fire(g0 + 2, 0)

        drain(1)
        combine_store(g0 + 1, 1, first)
        return carry

    lax.fori_loop(0, NCH // 2, body, 0, unroll=False)
    for b in range(2):
        pltpu.make_async_copy(out_v.at[b], out_hbm.at[pl.ds(base_pt, K)],
                              sems_out[b]).wait()


@functools.partial(
    pl.kernel,
    out_type=jax.ShapeDtypeStruct((NPTS, PROJ), jnp.float32),
    mesh=plsc.VectorSubcoreMesh(core_axis_name="c", subcore_axis_name="s"),
    scratch_types=[
        pltpu.VMEM((PPW,), jnp.float32),
        pltpu.VMEM((PPW,), jnp.float32),
        pltpu.VMEM((PPW,), jnp.float32),
        pltpu.VMEM((2, 8, K), jnp.int32),
        pltpu.VMEM((2, 8, K), jnp.float32),
        pltpu.VMEM((2, 8, K, PROJ), jnp.float32),
        pltpu.VMEM((2, K, PROJ), jnp.float32),
        pltpu.SemaphoreType.DMA,
        pltpu.SemaphoreType.DMA,
        pltpu.SemaphoreType.DMA,
        pltpu.SemaphoreType.DMA,
    ],
)
def _trilerp_sc(cx_hbm, cy_hbm, cz_hbm, table_hbm, out_hbm, *scratch):
    _sc_body(cx_hbm, cy_hbm, cz_hbm, table_hbm, out_hbm, *scratch)


def kernel(coordinates, pos_enc):
    B, N, _ = coordinates.shape
    ct = coordinates.reshape(B * N, 3)
    cx, cy, cz = ct[:, 2], ct[:, 1], ct[:, 0]
    table = jnp.transpose(pos_enc[0], (1, 2, 3, 0)).reshape(GRID ** 3, PROJ)
    out = _trilerp_sc(cx, cy, cz, table)
    return out.reshape(B, N, PROJ)
